# Initial kernel scaffold; baseline (speedup 1.0000x reference)
import functools

import jax
import jax.numpy as jnp
from jax import lax
from jax.experimental import pallas as pl
from jax.experimental.pallas import tpu as pltpu
from jax.experimental.pallas import tpu_sc as plsc

N = 10000
E = 320000
D = 128
G = 64
C = 10

NTILES = 32
EDGES_PER_TILE = E // NTILES
K = 100
CH = EDGES_PER_TILE // K
NPAD = 10240
RPT = NPAD // 16
ZR = 128

BM = 1000


def _agg_body(with_deg):
    def body(table, src_hbm, dst_hbm, *refs):
        if with_deg:
            (out_acc, out_deg, src_v, dst_v, rows_v, zbuf, acc_sh, sem,
             ones_v, zdeg, deg_sh) = refs
        else:
            out_acc, src_v, dst_v, rows_v, zbuf, acc_sh, sem = refs
        cid = lax.axis_index("c")
        sid = lax.axis_index("s")
        wid = sid * 2 + cid

        pltpu.sync_copy(src_hbm.at[wid], src_v)
        pltpu.sync_copy(dst_hbm.at[wid], dst_v)

        def fill(i, _):
            for j in range(D // 16):
                zbuf[i, pl.ds(j * 16, 16)] = jnp.zeros((16,), jnp.float32)
            return 0
        lax.fori_loop(0, ZR, fill, 0)
        if with_deg:
            def fill1(i, _):
                zdeg[i, pl.ds(0, 16)] = jnp.zeros((16,), jnp.float32)
                return 0
            lax.fori_loop(0, ZR, fill1, 0)
            def fillo(i, _):
                ones_v[i, pl.ds(0, 16)] = jnp.ones((16,), jnp.float32)
                return 0
            lax.fori_loop(0, K, fillo, 0)

        for j in range(RPT // ZR):
            rows = pl.ds(sid * RPT + j * ZR, ZR)
            pltpu.sync_copy(zbuf, acc_sh.at[rows])
            if with_deg:
                pltpu.sync_copy(zdeg, deg_sh.at[rows])
        plsc.subcore_barrier()

        def chunk(c, _):
            pltpu.async_copy(table.at[src_v.at[c]], rows_v, sem).wait()
            pltpu.sync_copy(rows_v, acc_sh.at[dst_v.at[c]], add=True)
            if with_deg:
                pltpu.sync_copy(ones_v, deg_sh.at[dst_v.at[c]], add=True)
            return 0
        lax.fori_loop(0, CH, chunk, 0)
        plsc.subcore_barrier()

        rows = pl.ds(sid * RPT, RPT)
        pltpu.sync_copy(acc_sh.at[rows], out_acc.at[cid, rows])
        if with_deg:
            pltpu.sync_copy(deg_sh.at[rows], out_deg.at[cid, rows])
    return body


def _make_agg(with_deg):
    mesh = plsc.VectorSubcoreMesh(core_axis_name="c", subcore_axis_name="s")
    out_type = [jax.ShapeDtypeStruct((2, NPAD, D), jnp.float32)]
    scratch = [
        pltpu.VMEM((CH, K), jnp.int32),
        pltpu.VMEM((CH, K), jnp.int32),
        pltpu.VMEM((K, D), jnp.float32),
        pltpu.VMEM((ZR, D), jnp.float32),
        pltpu.VMEM_SHARED((NPAD, D), jnp.float32),
        pltpu.SemaphoreType.DMA,
    ]
    if with_deg:
        out_type.append(jax.ShapeDtypeStruct((2, NPAD, 16), jnp.float32))
        scratch += [
            pltpu.VMEM((K, 16), jnp.float32),
            pltpu.VMEM((ZR, 16), jnp.float32),
            pltpu.VMEM_SHARED((NPAD, 16), jnp.float32),
        ]
    return pl.kernel(_agg_body(with_deg), out_type=out_type, mesh=mesh,
                     scratch_types=scratch)


_agg_deg = _make_agg(True)
_agg = _make_agg(False)


def _dotT(a, b):
    return lax.dot_general(a, b, (((1,), (1,)), ((), ())),
                           preferred_element_type=jnp.float32)


def _mmT_body(x_ref, w_ref, o_ref):
    o_ref[...] = _dotT(x_ref[...], w_ref[...])


_mmT = pl.pallas_call(
    _mmT_body,
    grid=(N // BM,),
    in_specs=[pl.BlockSpec((BM, D), lambda i: (i, 0)),
              pl.BlockSpec((D, D), lambda i: (0, 0))],
    out_specs=pl.BlockSpec((BM, D), lambda i: (i, 0)),
    out_shape=jax.ShapeDtypeStruct((N, D), jnp.float32),
)


def _mid_body(acc_ref, deg_ref, x_ref, wr1_ref, bl1_ref, wl2_ref, wr2_ref,
              xl2_ref, xr2_ref):
    acc = acc_ref[0] + acc_ref[1]
    deg = deg_ref[0, :, 0:1] + deg_ref[1, :, 0:1]
    mean = acc / jnp.maximum(deg, 1.0)
    h = jnp.maximum(mean + bl1_ref[...] + _dotT(x_ref[...], wr1_ref[...]), 0.0)
    xl2_ref[...] = _dotT(h, wl2_ref[...])
    xr2_ref[...] = _dotT(h, wr2_ref[...])


_mid = pl.pallas_call(
    _mid_body,
    grid=(N // BM,),
    in_specs=[pl.BlockSpec((2, BM, D), lambda i: (0, i, 0)),
              pl.BlockSpec((2, BM, 16), lambda i: (0, i, 0)),
              pl.BlockSpec((BM, D), lambda i: (i, 0)),
              pl.BlockSpec((D, D), lambda i: (0, 0)),
              pl.BlockSpec((1, D), lambda i: (0, 0)),
              pl.BlockSpec((D, D), lambda i: (0, 0)),
              pl.BlockSpec((D, D), lambda i: (0, 0))],
    out_specs=[pl.BlockSpec((BM, D), lambda i: (i, 0)),
               pl.BlockSpec((BM, D), lambda i: (i, 0))],
    out_shape=[jax.ShapeDtypeStruct((N, D), jnp.float32),
               jax.ShapeDtypeStruct((N, D), jnp.float32)],
)


def _head_body(acc_ref, deg_ref, xr2_ref, bl2_ref, batch_ref, wc_ref, bc_ref,
               out_ref, pool_ref):
    i = pl.program_id(0)
    acc = acc_ref[0] + acc_ref[1]
    deg = deg_ref[0, :, 0:1] + deg_ref[1, :, 0:1]
    h = jnp.maximum(acc / jnp.maximum(deg, 1.0) + bl2_ref[...] + xr2_ref[...],
                    0.0)
    onehot = (batch_ref[...] ==
              lax.broadcasted_iota(jnp.int32, (BM, G), 1)).astype(jnp.float32)
    part = lax.dot_general(onehot, h, (((0,), (0,)), ((), ())),
                           preferred_element_type=jnp.float32)

    @pl.when(i == 0)
    def _():
        pool_ref[...] = part

    @pl.when(i > 0)
    def _():
        pool_ref[...] += part

    @pl.when(i == pl.num_programs(0) - 1)
    def _():
        out_ref[...] = jax.nn.sigmoid(_dotT(pool_ref[...], wc_ref[...])
                                      + bc_ref[...])


_head = pl.pallas_call(
    _head_body,
    grid=(N // BM,),
    in_specs=[pl.BlockSpec((2, BM, D), lambda i: (0, i, 0)),
              pl.BlockSpec((2, BM, 16), lambda i: (0, i, 0)),
              pl.BlockSpec((BM, D), lambda i: (i, 0)),
              pl.BlockSpec((1, D), lambda i: (0, 0)),
              pl.BlockSpec((BM, 1), lambda i: (i, 0)),
              pl.BlockSpec((C, D), lambda i: (0, 0)),
              pl.BlockSpec((1, C), lambda i: (0, 0))],
    out_specs=pl.BlockSpec((G, C), lambda i: (0, 0)),
    out_shape=jax.ShapeDtypeStruct((G, C), jnp.float32),
    scratch_shapes=[pltpu.VMEM((G, D), jnp.float32)],
)


@jax.jit
def kernel(x, edge_index, batch, Wl1, bl1, Wr1, Wl2, bl2, Wr2, Wc, bc):
    src3 = edge_index[0].reshape(NTILES, CH, K)
    dst3 = edge_index[1].reshape(NTILES, CH, K)

    xl1 = _mmT(x, Wl1)
    acc1, degp = _agg_deg(xl1, src3, dst3)
    xl2, xr2 = _mid(acc1, degp, x, Wr1, bl1.reshape(1, D), Wl2, Wr2)
    acc2 = _agg(xl2, src3, dst3)
    out = _head(acc2, degp, xr2, bl2.reshape(1, D), batch.reshape(N, 1),
                Wc, bc.reshape(1, C))
    return out


# speedup vs baseline: 6.1143x; 6.1143x over previous
import jax
import jax.numpy as jnp
from jax import lax
from jax.experimental import pallas as pl
from jax.experimental.pallas import tpu as pltpu
from jax.experimental.pallas import tpu_sc as plsc

N = 10000
E = 320000
D = 128
HD = D // 2
G = 64
C = 10

NT = 16
EPT = E // NT
K = 100
CH = EPT // K
NPAD = 10240
RPT = NPAD // NT
ZR = 128

BM = 1000


def _agg_body(with_deg):
    def body(table, src_hbm, dst_hbm, *refs):
        if with_deg:
            (out_acc, out_deg, src_v, dst_v, rows_v, zbuf, acc_sh, sem,
             ones_v, zdeg, deg_sh) = refs
        else:
            out_acc, src_v, dst_v, rows_v, zbuf, acc_sh, sem = refs
        cid = lax.axis_index("c")
        sid = lax.axis_index("s")

        pltpu.sync_copy(src_hbm.at[sid], src_v)
        pltpu.sync_copy(dst_hbm.at[sid], dst_v)

        def fill(i, _):
            for j in range(HD // 16):
                zbuf[i, pl.ds(j * 16, 16)] = jnp.zeros((16,), jnp.float32)
            return 0
        lax.fori_loop(0, ZR, fill, 0)
        if with_deg:
            def fill1(i, _):
                zdeg[i, pl.ds(0, 16)] = jnp.zeros((16,), jnp.float32)
                return 0
            lax.fori_loop(0, ZR, fill1, 0)
            def fillo(i, _):
                ones_v[i, pl.ds(0, 16)] = jnp.ones((16,), jnp.float32)
                return 0
            lax.fori_loop(0, K, fillo, 0)

        for j in range(RPT // ZR):
            rows = pl.ds(sid * RPT + j * ZR, ZR)
            pltpu.sync_copy(zbuf, acc_sh.at[rows])
            if with_deg:
                pltpu.sync_copy(zdeg, deg_sh.at[rows])
        plsc.subcore_barrier()

        def chunk(c, _):
            pltpu.async_copy(table.at[cid].at[src_v.at[c]], rows_v, sem).wait()
            pltpu.sync_copy(rows_v, acc_sh.at[dst_v.at[c]], add=True)
            if with_deg:
                @pl.when(cid == 0)
                def _():
                    pltpu.sync_copy(ones_v, deg_sh.at[dst_v.at[c]], add=True)
            return 0
        lax.fori_loop(0, CH, chunk, 0)
        plsc.subcore_barrier()

        rows = pl.ds(sid * RPT, RPT)
        pltpu.sync_copy(acc_sh.at[rows], out_acc.at[cid, rows])
        if with_deg:
            @pl.when(cid == 0)
            def _():
                pltpu.sync_copy(deg_sh.at[rows], out_deg.at[rows])
    return body


def _make_agg(with_deg):
    mesh = plsc.VectorSubcoreMesh(core_axis_name="c", subcore_axis_name="s")
    acc_t = jax.ShapeDtypeStruct((2, NPAD, HD), jnp.float32)
    out_type = acc_t
    scratch = [
        pltpu.VMEM((CH, K), jnp.int32),
        pltpu.VMEM((CH, K), jnp.int32),
        pltpu.VMEM((K, HD), jnp.float32),
        pltpu.VMEM((ZR, HD), jnp.float32),
        pltpu.VMEM_SHARED((NPAD, HD), jnp.float32),
        pltpu.SemaphoreType.DMA,
    ]
    if with_deg:
        out_type = [acc_t, jax.ShapeDtypeStruct((NPAD, 16), jnp.float32)]
        scratch += [
            pltpu.VMEM((K, 16), jnp.float32),
            pltpu.VMEM((ZR, 16), jnp.float32),
            pltpu.VMEM_SHARED((NPAD, 16), jnp.float32),
        ]
    return pl.kernel(_agg_body(with_deg), out_type=out_type, mesh=mesh,
                     scratch_types=scratch,
                     compiler_params=pltpu.CompilerParams(
                         use_tc_tiling_on_sc=False))


_agg_deg = _make_agg(True)
_agg = _make_agg(False)


def _dotT(a, b):
    return lax.dot_general(a, b, (((1,), (1,)), ((), ())),
                           preferred_element_type=jnp.float32)


def _mmT_body(x_ref, w_ref, o_ref):
    o_ref[0] = _dotT(x_ref[...], w_ref[0:HD, :])
    o_ref[1] = _dotT(x_ref[...], w_ref[HD:D, :])


_mmT = pl.pallas_call(
    _mmT_body,
    grid=(N // BM,),
    in_specs=[pl.BlockSpec((BM, D), lambda i: (i, 0)),
              pl.BlockSpec((D, D), lambda i: (0, 0))],
    out_specs=pl.BlockSpec((2, BM, HD), lambda i: (0, i, 0)),
    out_shape=jax.ShapeDtypeStruct((2, N, HD), jnp.float32),
)


def _mid_body(acc_ref, deg_ref, x_ref, wr1_ref, bl1_ref, wl2_ref, wr2_ref,
              xl2_ref, xr2_ref):
    acc = jnp.concatenate([acc_ref[0], acc_ref[1]], axis=1)
    deg = deg_ref[:, 0:1]
    mean = acc / jnp.maximum(deg, 1.0)
    h = jnp.maximum(mean + bl1_ref[...] + _dotT(x_ref[...], wr1_ref[...]), 0.0)
    xl2_ref[0] = _dotT(h, wl2_ref[0:HD, :])
    xl2_ref[1] = _dotT(h, wl2_ref[HD:D, :])
    xr2_ref[...] = _dotT(h, wr2_ref[...])


_mid = pl.pallas_call(
    _mid_body,
    grid=(N // BM,),
    in_specs=[pl.BlockSpec((2, BM, HD), lambda i: (0, i, 0)),
              pl.BlockSpec((BM, 16), lambda i: (i, 0)),
              pl.BlockSpec((BM, D), lambda i: (i, 0)),
              pl.BlockSpec((D, D), lambda i: (0, 0)),
              pl.BlockSpec((1, D), lambda i: (0, 0)),
              pl.BlockSpec((D, D), lambda i: (0, 0)),
              pl.BlockSpec((D, D), lambda i: (0, 0))],
    out_specs=[pl.BlockSpec((2, BM, HD), lambda i: (0, i, 0)),
               pl.BlockSpec((BM, D), lambda i: (i, 0))],
    out_shape=[jax.ShapeDtypeStruct((2, N, HD), jnp.float32),
               jax.ShapeDtypeStruct((N, D), jnp.float32)],
)


def _head_body(acc_ref, deg_ref, xr2_ref, bl2_ref, batch_ref, wc_ref, bc_ref,
               out_ref, pool_ref):
    i = pl.program_id(0)
    acc = jnp.concatenate([acc_ref[0], acc_ref[1]], axis=1)
    deg = deg_ref[:, 0:1]
    h = jnp.maximum(acc / jnp.maximum(deg, 1.0) + bl2_ref[...] + xr2_ref[...],
                    0.0)
    onehot = (batch_ref[...] ==
              lax.broadcasted_iota(jnp.int32, (BM, G), 1)).astype(jnp.float32)
    part = lax.dot_general(onehot, h, (((0,), (0,)), ((), ())),
                           preferred_element_type=jnp.float32)

    @pl.when(i == 0)
    def _():
        pool_ref[...] = part

    @pl.when(i > 0)
    def _():
        pool_ref[...] += part

    @pl.when(i == pl.num_programs(0) - 1)
    def _():
        out_ref[...] = jax.nn.sigmoid(_dotT(pool_ref[...], wc_ref[...])
                                      + bc_ref[...])


_head = pl.pallas_call(
    _head_body,
    grid=(N // BM,),
    in_specs=[pl.BlockSpec((2, BM, HD), lambda i: (0, i, 0)),
              pl.BlockSpec((BM, 16), lambda i: (i, 0)),
              pl.BlockSpec((BM, D), lambda i: (i, 0)),
              pl.BlockSpec((1, D), lambda i: (0, 0)),
              pl.BlockSpec((BM, 1), lambda i: (i, 0)),
              pl.BlockSpec((C, D), lambda i: (0, 0)),
              pl.BlockSpec((1, C), lambda i: (0, 0))],
    out_specs=pl.BlockSpec((G, C), lambda i: (0, 0)),
    out_shape=jax.ShapeDtypeStruct((G, C), jnp.float32),
    scratch_shapes=[pltpu.VMEM((G, D), jnp.float32)],
)


@jax.jit
def kernel(x, edge_index, batch, Wl1, bl1, Wr1, Wl2, bl2, Wr2, Wc, bc):
    src3 = edge_index[0].reshape(NT, CH, K)
    dst3 = edge_index[1].reshape(NT, CH, K)

    xl1 = _mmT(x, Wl1)
    acc1, degp = _agg_deg(xl1, src3, dst3)
    xl2, xr2 = _mid(acc1, degp, x, Wr1, bl1.reshape(1, D), Wl2, Wr2)
    acc2 = _agg(xl2, src3, dst3)
    out = _head(acc2, degp, xr2, bl2.reshape(1, D), batch.reshape(N, 1),
                Wc, bc.reshape(1, C))
    return out


# speedup vs baseline: 10.5835x; 1.7309x over previous
import jax
import jax.numpy as jnp
from jax import lax
from jax.experimental import pallas as pl
from jax.experimental.pallas import tpu as pltpu
from jax.experimental.pallas import tpu_sc as plsc

N = 10000
E = 320000
D = 128
HD = D // 2
G = 64
C = 10

NT = 16
EPT = E // NT
K = 125
CH = EPT // K
HCH = CH // 2
NPAD = 10240
RPT = NPAD // NT
ZR = 128

BM = 1000


def _agg_body(with_deg):
    def body(table, src_hbm, dst_hbm, *refs):
        if with_deg:
            (out_acc, out_deg, src_v, dst_v, rows_v, zbuf, acc_sh, sems,
             ones_v, zdeg, deg_sh) = refs
        else:
            out_acc, src_v, dst_v, rows_v, zbuf, acc_sh, sems = refs
        cid = lax.axis_index("c")
        sid = lax.axis_index("s")

        pltpu.sync_copy(src_hbm.at[sid], src_v)
        pltpu.sync_copy(dst_hbm.at[sid], dst_v)

        def fill(i, _):
            for j in range(HD // 16):
                zbuf[i, pl.ds(j * 16, 16)] = jnp.zeros((16,), jnp.float32)
            return 0
        lax.fori_loop(0, ZR, fill, 0)
        if with_deg:
            def fill1(i, _):
                zdeg[i, pl.ds(0, 16)] = jnp.zeros((16,), jnp.float32)
                return 0
            lax.fori_loop(0, ZR, fill1, 0)
            def fillo(i, _):
                ones_v[i, pl.ds(0, 16)] = jnp.ones((16,), jnp.float32)
                return 0
            lax.fori_loop(0, K, fillo, 0)

        for j in range(RPT // ZR):
            rows = pl.ds(sid * RPT + j * ZR, ZR)
            pltpu.sync_copy(zbuf, acc_sh.at[rows])
            if with_deg:
                pltpu.sync_copy(zdeg, deg_sh.at[rows])
        plsc.subcore_barrier()

        for b in range(2):
            pltpu.async_copy(table.at[cid].at[src_v.at[b]], rows_v.at[b],
                             sems.at[b])

        def chunk(c, _):
            for b in range(2):
                g = 2 * c + b
                if with_deg:
                    @pl.when(g // HCH == cid)
                    def _():
                        pltpu.sync_copy(ones_v, deg_sh.at[dst_v.at[g]],
                                        add=True)
                pltpu.make_async_copy(table.at[cid].at[src_v.at[g]],
                                      rows_v.at[b], sems.at[b]).wait()
                pltpu.sync_copy(rows_v.at[b], acc_sh.at[dst_v.at[g]], add=True)
                nxt = g + 2

                @pl.when(nxt < CH)
                def _():
                    pltpu.async_copy(table.at[cid].at[src_v.at[nxt]],
                                     rows_v.at[b], sems.at[b])
            return 0
        lax.fori_loop(0, CH // 2, chunk, 0)
        plsc.subcore_barrier()

        rows = pl.ds(sid * RPT, RPT)
        pltpu.sync_copy(acc_sh.at[rows], out_acc.at[cid, rows])
        if with_deg:
            pltpu.sync_copy(deg_sh.at[rows], out_deg.at[cid, rows])
    return body


def _make_agg(with_deg):
    mesh = plsc.VectorSubcoreMesh(core_axis_name="c", subcore_axis_name="s")
    acc_t = jax.ShapeDtypeStruct((2, NPAD, HD), jnp.float32)
    out_type = acc_t
    scratch = [
        pltpu.VMEM((CH, K), jnp.int32),
        pltpu.VMEM((CH, K), jnp.int32),
        pltpu.VMEM((2, K, HD), jnp.float32),
        pltpu.VMEM((ZR, HD), jnp.float32),
        pltpu.VMEM_SHARED((NPAD, HD), jnp.float32),
        pltpu.SemaphoreType.DMA((2,)),
    ]
    if with_deg:
        out_type = [acc_t, jax.ShapeDtypeStruct((2, NPAD, 16), jnp.float32)]
        scratch += [
            pltpu.VMEM((K, 16), jnp.float32),
            pltpu.VMEM((ZR, 16), jnp.float32),
            pltpu.VMEM_SHARED((NPAD, 16), jnp.float32),
        ]
    return pl.kernel(_agg_body(with_deg), out_type=out_type, mesh=mesh,
                     scratch_types=scratch,
                     compiler_params=pltpu.CompilerParams(
                         use_tc_tiling_on_sc=False))


_agg_deg = _make_agg(True)
_agg = _make_agg(False)


def _dotT(a, b):
    return lax.dot_general(a, b, (((1,), (1,)), ((), ())),
                           preferred_element_type=jnp.float32)


def _mmT_body(x_ref, w_ref, o_ref):
    o_ref[0] = _dotT(x_ref[...], w_ref[0:HD, :])
    o_ref[1] = _dotT(x_ref[...], w_ref[HD:D, :])


_mmT = pl.pallas_call(
    _mmT_body,
    grid=(N // BM,),
    in_specs=[pl.BlockSpec((BM, D), lambda i: (i, 0)),
              pl.BlockSpec((D, D), lambda i: (0, 0))],
    out_specs=pl.BlockSpec((2, BM, HD), lambda i: (0, i, 0)),
    out_shape=jax.ShapeDtypeStruct((2, N, HD), jnp.float32),
)


def _mid_body(acc_ref, deg_ref, x_ref, wr1_ref, bl1_ref, wl2_ref, wr2_ref,
              xl2_ref, xr2_ref):
    acc = jnp.concatenate([acc_ref[0], acc_ref[1]], axis=1)
    deg = deg_ref[0, :, 0:1] + deg_ref[1, :, 0:1]
    mean = acc / jnp.maximum(deg, 1.0)
    h = jnp.maximum(mean + bl1_ref[...] + _dotT(x_ref[...], wr1_ref[...]), 0.0)
    xl2_ref[0] = _dotT(h, wl2_ref[0:HD, :])
    xl2_ref[1] = _dotT(h, wl2_ref[HD:D, :])
    xr2_ref[...] = _dotT(h, wr2_ref[...])


_mid = pl.pallas_call(
    _mid_body,
    grid=(N // BM,),
    in_specs=[pl.BlockSpec((2, BM, HD), lambda i: (0, i, 0)),
              pl.BlockSpec((2, BM, 16), lambda i: (0, i, 0)),
              pl.BlockSpec((BM, D), lambda i: (i, 0)),
              pl.BlockSpec((D, D), lambda i: (0, 0)),
              pl.BlockSpec((1, D), lambda i: (0, 0)),
              pl.BlockSpec((D, D), lambda i: (0, 0)),
              pl.BlockSpec((D, D), lambda i: (0, 0))],
    out_specs=[pl.BlockSpec((2, BM, HD), lambda i: (0, i, 0)),
               pl.BlockSpec((BM, D), lambda i: (i, 0))],
    out_shape=[jax.ShapeDtypeStruct((2, N, HD), jnp.float32),
               jax.ShapeDtypeStruct((N, D), jnp.float32)],
)


def _head_body(acc_ref, deg_ref, xr2_ref, bl2_ref, batch_ref, wc_ref, bc_ref,
               out_ref, pool_ref):
    i = pl.program_id(0)
    acc = jnp.concatenate([acc_ref[0], acc_ref[1]], axis=1)
    deg = deg_ref[0, :, 0:1] + deg_ref[1, :, 0:1]
    h = jnp.maximum(acc / jnp.maximum(deg, 1.0) + bl2_ref[...] + xr2_ref[...],
                    0.0)
    onehot = (batch_ref[...] ==
              lax.broadcasted_iota(jnp.int32, (BM, G), 1)).astype(jnp.float32)
    part = lax.dot_general(onehot, h, (((0,), (0,)), ((), ())),
                           preferred_element_type=jnp.float32,
                           precision=lax.Precision.HIGHEST)

    @pl.when(i == 0)
    def _():
        pool_ref[...] = part

    @pl.when(i > 0)
    def _():
        pool_ref[...] += part

    @pl.when(i == pl.num_programs(0) - 1)
    def _():
        out_ref[...] = jax.nn.sigmoid(_dotT(pool_ref[...], wc_ref[...])
                                      + bc_ref[...])


_head = pl.pallas_call(
    _head_body,
    grid=(N // BM,),
    in_specs=[pl.BlockSpec((2, BM, HD), lambda i: (0, i, 0)),
              pl.BlockSpec((2, BM, 16), lambda i: (0, i, 0)),
              pl.BlockSpec((BM, D), lambda i: (i, 0)),
              pl.BlockSpec((1, D), lambda i: (0, 0)),
              pl.BlockSpec((BM, 1), lambda i: (i, 0)),
              pl.BlockSpec((C, D), lambda i: (0, 0)),
              pl.BlockSpec((1, C), lambda i: (0, 0))],
    out_specs=pl.BlockSpec((G, C), lambda i: (0, 0)),
    out_shape=jax.ShapeDtypeStruct((G, C), jnp.float32),
    scratch_shapes=[pltpu.VMEM((G, D), jnp.float32)],
)


@jax.jit
def kernel(x, edge_index, batch, Wl1, bl1, Wr1, Wl2, bl2, Wr2, Wc, bc):
    src3 = edge_index[0].reshape(NT, CH, K)
    dst3 = edge_index[1].reshape(NT, CH, K)

    xl1 = _mmT(x, Wl1)
    acc1, degp = _agg_deg(xl1, src3, dst3)
    xl2, xr2 = _mid(acc1, degp, x, Wr1, bl1.reshape(1, D), Wl2, Wr2)
    acc2 = _agg(xl2, src3, dst3)
    out = _head(acc2, degp, xr2, bl2.reshape(1, D), batch.reshape(N, 1),
                Wc, bc.reshape(1, C))
    return out


# speedup vs baseline: 10.9574x; 1.0353x over previous
import jax
import jax.numpy as jnp
from jax import lax
from jax.experimental import pallas as pl
from jax.experimental.pallas import tpu as pltpu
from jax.experimental.pallas import tpu_sc as plsc

N = 10000
E = 320000
D = 128
HD = D // 2
G = 64
C = 10

NT = 16
EPT = E // NT
K = 125
CH = EPT // K
HCH = CH // 2
NPAD = 10240
RPT = NPAD // NT
ZR = 32

BM = 1000


def _agg_body(with_deg):
    def body(table, edges_hbm, *refs):
        if with_deg:
            (out_acc, out_deg, src_v, dst_v, rows_v, zbuf, acc_sh, sems_g,
             sems_s, ones_v, zdeg, deg_sh) = refs
        else:
            (out_acc, src_v, dst_v, rows_v, zbuf, acc_sh, sems_g,
             sems_s) = refs
        cid = lax.axis_index("c")
        sid = lax.axis_index("s")

        pltpu.sync_copy(edges_hbm.at[0, sid], src_v)
        pltpu.sync_copy(edges_hbm.at[1, sid], dst_v)

        def fill(i, _):
            for j in range(HD // 16):
                zbuf[i, pl.ds(j * 16, 16)] = jnp.zeros((16,), jnp.float32)
            return 0
        lax.fori_loop(0, ZR, fill, 0)
        if with_deg:
            def fill1(i, _):
                zdeg[i, pl.ds(0, 16)] = jnp.zeros((16,), jnp.float32)
                return 0
            lax.fori_loop(0, ZR, fill1, 0)
            def fillo(i, _):
                ones_v[i, pl.ds(0, 16)] = jnp.ones((16,), jnp.float32)
                return 0
            lax.fori_loop(0, K, fillo, 0)

        for j in range(RPT // ZR):
            rows = pl.ds(sid * RPT + j * ZR, ZR)
            pltpu.sync_copy(zbuf, acc_sh.at[rows])
            if with_deg:
                pltpu.sync_copy(zdeg, deg_sh.at[rows])
        plsc.subcore_barrier()

        for b in range(2):
            pltpu.async_copy(table.at[cid].at[src_v.at[b]], rows_v.at[b],
                             sems_g.at[b])

        def chunk(c, _):
            for b in range(4):
                g = 4 * c + b
                if with_deg:
                    @pl.when(g // HCH == cid)
                    def _():
                        pltpu.sync_copy(ones_v, deg_sh.at[dst_v.at[g]],
                                        add=True)
                pltpu.make_async_copy(table.at[cid].at[src_v.at[g]],
                                      rows_v.at[b], sems_g.at[b]).wait()
                pltpu.async_copy(rows_v.at[b], acc_sh.at[dst_v.at[g]],
                                 sems_s.at[b], add=True)
                nxt = g + 2
                bn = (b + 2) % 4

                @pl.when(nxt < CH)
                def _():
                    @pl.when(g >= 2)
                    def _():
                        pltpu.make_async_copy(rows_v.at[bn],
                                              acc_sh.at[dst_v.at[g]],
                                              sems_s.at[bn]).wait()
                    pltpu.async_copy(table.at[cid].at[src_v.at[nxt]],
                                     rows_v.at[bn], sems_g.at[bn])
            return 0
        lax.fori_loop(0, CH // 4, chunk, 0)
        for b in range(4):
            pltpu.make_async_copy(rows_v.at[b], acc_sh.at[dst_v.at[b]],
                                  sems_s.at[b]).wait()
        plsc.subcore_barrier()

        rows = pl.ds(sid * RPT, RPT)
        pltpu.sync_copy(acc_sh.at[rows], out_acc.at[cid, rows])
        if with_deg:
            pltpu.sync_copy(deg_sh.at[rows], out_deg.at[cid, rows])
    return body


def _make_agg(with_deg):
    mesh = plsc.VectorSubcoreMesh(core_axis_name="c", subcore_axis_name="s")
    acc_t = jax.ShapeDtypeStruct((2, NPAD, HD), jnp.float32)
    out_type = acc_t
    scratch = [
        pltpu.VMEM((CH, K), jnp.int32),
        pltpu.VMEM((CH, K), jnp.int32),
        pltpu.VMEM((4, K, HD), jnp.float32),
        pltpu.VMEM((ZR, HD), jnp.float32),
        pltpu.VMEM_SHARED((NPAD, HD), jnp.float32),
        pltpu.SemaphoreType.DMA((4,)),
        pltpu.SemaphoreType.DMA((4,)),
    ]
    if with_deg:
        out_type = [acc_t, jax.ShapeDtypeStruct((2, NPAD, 16), jnp.float32)]
        scratch += [
            pltpu.VMEM((K, 16), jnp.float32),
            pltpu.VMEM((ZR, 16), jnp.float32),
            pltpu.VMEM_SHARED((NPAD, 16), jnp.float32),
        ]
    return pl.kernel(_agg_body(with_deg), out_type=out_type, mesh=mesh,
                     scratch_types=scratch,
                     compiler_params=pltpu.CompilerParams(
                         use_tc_tiling_on_sc=False))


_agg_deg = _make_agg(True)
_agg = _make_agg(False)


def _cp_body(e_ref, o_ref):
    o_ref[...] = e_ref[...]


_cp = pl.pallas_call(
    _cp_body,
    grid=(NT,),
    in_specs=[pl.BlockSpec((2, 1, CH, K), lambda i: (0, i, 0, 0))],
    out_specs=pl.BlockSpec((2, 1, CH, K), lambda i: (0, i, 0, 0)),
    out_shape=jax.ShapeDtypeStruct((2, NT, CH, K), jnp.int32),
)


def _dotT(a, b):
    return lax.dot_general(a, b, (((1,), (1,)), ((), ())),
                           preferred_element_type=jnp.float32)


def _mmT_body(x_ref, w_ref, o_ref):
    o_ref[0] = _dotT(x_ref[...], w_ref[0:HD, :])
    o_ref[1] = _dotT(x_ref[...], w_ref[HD:D, :])


_mmT = pl.pallas_call(
    _mmT_body,
    grid=(N // BM,),
    in_specs=[pl.BlockSpec((BM, D), lambda i: (i, 0)),
              pl.BlockSpec((D, D), lambda i: (0, 0))],
    out_specs=pl.BlockSpec((2, BM, HD), lambda i: (0, i, 0)),
    out_shape=jax.ShapeDtypeStruct((2, N, HD), jnp.float32),
)


def _mid_body(acc_ref, deg_ref, x_ref, wr1_ref, bl1_ref, wl2_ref, wr2_ref,
              xl2_ref, xr2_ref):
    acc = jnp.concatenate([acc_ref[0], acc_ref[1]], axis=1)
    deg = deg_ref[0, :, 0:1] + deg_ref[1, :, 0:1]
    mean = acc / jnp.maximum(deg, 1.0)
    h = jnp.maximum(mean + bl1_ref[...] + _dotT(x_ref[...], wr1_ref[...]), 0.0)
    xl2_ref[0] = _dotT(h, wl2_ref[0:HD, :])
    xl2_ref[1] = _dotT(h, wl2_ref[HD:D, :])
    xr2_ref[...] = _dotT(h, wr2_ref[...])


_mid = pl.pallas_call(
    _mid_body,
    grid=(N // BM,),
    in_specs=[pl.BlockSpec((2, BM, HD), lambda i: (0, i, 0)),
              pl.BlockSpec((2, BM, 16), lambda i: (0, i, 0)),
              pl.BlockSpec((BM, D), lambda i: (i, 0)),
              pl.BlockSpec((D, D), lambda i: (0, 0)),
              pl.BlockSpec((1, D), lambda i: (0, 0)),
              pl.BlockSpec((D, D), lambda i: (0, 0)),
              pl.BlockSpec((D, D), lambda i: (0, 0))],
    out_specs=[pl.BlockSpec((2, BM, HD), lambda i: (0, i, 0)),
               pl.BlockSpec((BM, D), lambda i: (i, 0))],
    out_shape=[jax.ShapeDtypeStruct((2, N, HD), jnp.float32),
               jax.ShapeDtypeStruct((N, D), jnp.float32)],
)


def _head_body(acc_ref, deg_ref, xr2_ref, bl2_ref, batch_ref, wc_ref, bc_ref,
               out_ref, pool_ref):
    i = pl.program_id(0)
    acc = jnp.concatenate([acc_ref[0], acc_ref[1]], axis=1)
    deg = deg_ref[0, :, 0:1] + deg_ref[1, :, 0:1]
    h = jnp.maximum(acc / jnp.maximum(deg, 1.0) + bl2_ref[...] + xr2_ref[...],
                    0.0)
    onehot = (batch_ref[...] ==
              lax.broadcasted_iota(jnp.int32, (BM, G), 1)).astype(jnp.float32)
    part = lax.dot_general(onehot, h, (((0,), (0,)), ((), ())),
                           preferred_element_type=jnp.float32,
                           precision=lax.Precision.HIGHEST)

    @pl.when(i == 0)
    def _():
        pool_ref[...] = part

    @pl.when(i > 0)
    def _():
        pool_ref[...] += part

    @pl.when(i == pl.num_programs(0) - 1)
    def _():
        out_ref[...] = jax.nn.sigmoid(_dotT(pool_ref[...], wc_ref[...])
                                      + bc_ref[...])


_head = pl.pallas_call(
    _head_body,
    grid=(N // BM,),
    in_specs=[pl.BlockSpec((2, BM, HD), lambda i: (0, i, 0)),
              pl.BlockSpec((2, BM, 16), lambda i: (0, i, 0)),
              pl.BlockSpec((BM, D), lambda i: (i, 0)),
              pl.BlockSpec((1, D), lambda i: (0, 0)),
              pl.BlockSpec((BM, 1), lambda i: (i, 0)),
              pl.BlockSpec((C, D), lambda i: (0, 0)),
              pl.BlockSpec((1, C), lambda i: (0, 0))],
    out_specs=pl.BlockSpec((G, C), lambda i: (0, 0)),
    out_shape=jax.ShapeDtypeStruct((G, C), jnp.float32),
    scratch_shapes=[pltpu.VMEM((G, D), jnp.float32)],
)


@jax.jit
def kernel(x, edge_index, batch, Wl1, bl1, Wr1, Wl2, bl2, Wr2, Wc, bc):
    edges = _cp(edge_index.reshape(2, NT, CH, K))

    xl1 = _mmT(x, Wl1)
    acc1, degp = _agg_deg(xl1, edges)
    xl2, xr2 = _mid(acc1, degp, x, Wr1, bl1.reshape(1, D), Wl2, Wr2)
    acc2 = _agg(xl2, edges)
    out = _head(acc2, degp, xr2, bl2.reshape(1, D), batch.reshape(N, 1),
                Wc, bc.reshape(1, C))
    return out
